# Initial kernel scaffold; baseline (speedup 1.0000x reference)
import jax
import jax.numpy as jnp
from jax.experimental import pallas as pl


def kernel(x, edge_index, W1, b1, W2, b2, W3, b3):
    raise NotImplementedError("write your pallas kernel here")


# speedup vs baseline: 3.1597x; 3.1597x over previous
import functools
import jax
import jax.numpy as jnp
from jax import lax
from jax.experimental import pallas as pl
from jax.experimental.pallas import tpu as pltpu

N = 10000
D = 128
E = 320000
ROWS = 1000


def _first_layer_body(deg_ref, x_ref, w_ref, y_ref, dinv_ref):
    deg = deg_ref[...]
    dinv = lax.rsqrt(deg)
    dinv_ref[...] = dinv
    y_ref[...] = jnp.dot(x_ref[...] * dinv, w_ref[...],
                         preferred_element_type=jnp.float32)


def _first_layer(deg, x, W1):
    grid = (N // ROWS,)
    return pl.pallas_call(
        _first_layer_body,
        grid=grid,
        in_specs=[
            pl.BlockSpec((ROWS, 1), lambda i: (i, 0)),
            pl.BlockSpec((ROWS, D), lambda i: (i, 0)),
            pl.BlockSpec((D, D), lambda i: (0, 0)),
        ],
        out_specs=[
            pl.BlockSpec((ROWS, D), lambda i: (i, 0)),
            pl.BlockSpec((ROWS, 1), lambda i: (i, 0)),
        ],
        out_shape=[
            jax.ShapeDtypeStruct((N, D), jnp.float32),
            jax.ShapeDtypeStruct((N, 1), jnp.float32),
        ],
    )(deg, x, W1)


def _mid_layer_body(agg_ref, y_ref, dinv_ref, b_ref, w_ref, ynext_ref):
    dinv = dinv_ref[...]
    agg = agg_ref[0] + agg_ref[1] + y_ref[...]
    out = dinv * agg + b_ref[...]
    xn = jnp.where(out > 0, out, jnp.exp(jnp.minimum(out, 0.0)) - 1.0)
    ynext_ref[...] = jnp.dot(xn * dinv, w_ref[...],
                             preferred_element_type=jnp.float32)


def _mid_layer(agg, y, dinv, b, Wn):
    grid = (N // ROWS,)
    return pl.pallas_call(
        _mid_layer_body,
        grid=grid,
        in_specs=[
            pl.BlockSpec((2, ROWS, D), lambda i: (0, i, 0)),
            pl.BlockSpec((ROWS, D), lambda i: (i, 0)),
            pl.BlockSpec((ROWS, 1), lambda i: (i, 0)),
            pl.BlockSpec((1, D), lambda i: (0, 0)),
            pl.BlockSpec((D, D), lambda i: (0, 0)),
        ],
        out_specs=pl.BlockSpec((ROWS, D), lambda i: (i, 0)),
        out_shape=jax.ShapeDtypeStruct((N, D), jnp.float32),
    )(agg, y, dinv, b, Wn)


def _last_layer_body(agg_ref, y_ref, dinv_ref, b_ref, out_ref):
    agg = agg_ref[0] + agg_ref[1] + y_ref[...]
    out_ref[...] = dinv_ref[...] * agg + b_ref[...]


def _last_layer(agg, y, dinv, b):
    grid = (N // ROWS,)
    return pl.pallas_call(
        _last_layer_body,
        grid=grid,
        in_specs=[
            pl.BlockSpec((2, ROWS, D), lambda i: (0, i, 0)),
            pl.BlockSpec((ROWS, D), lambda i: (i, 0)),
            pl.BlockSpec((ROWS, 1), lambda i: (i, 0)),
            pl.BlockSpec((1, D), lambda i: (0, 0)),
        ],
        out_specs=pl.BlockSpec((ROWS, D), lambda i: (i, 0)),
        out_shape=jax.ShapeDtypeStruct((N, D), jnp.float32),
    )(agg, y, dinv, b)


def _aggregate(y, src, dst):
    half = E // 2
    p0 = jnp.zeros((N, D), jnp.float32).at[dst[:half]].add(y[src[:half]])
    p1 = jnp.zeros((N, D), jnp.float32).at[dst[half:]].add(y[src[half:]])
    return jnp.stack([p0, p1])


@jax.jit
def kernel(x, edge_index, W1, b1, W2, b2, W3, b3):
    src = edge_index[0]
    dst = edge_index[1]
    deg = (jnp.zeros((N,), jnp.float32).at[dst].add(1.0) + 1.0).reshape(N, 1)
    y1, dinv = _first_layer(deg, x, W1)
    a1 = _aggregate(y1, src, dst)
    y2 = _mid_layer(a1, y1, dinv, b1.reshape(1, D), W2)
    a2 = _aggregate(y2, src, dst)
    y3 = _mid_layer(a2, y2, dinv, b2.reshape(1, D), W3)
    a3 = _aggregate(y3, src, dst)
    return _last_layer(a3, y3, dinv, b3.reshape(1, D))


# speedup vs baseline: 11.1971x; 3.5438x over previous
import functools
import jax
import jax.numpy as jnp
from jax import lax
from jax.experimental import pallas as pl
from jax.experimental.pallas import tpu as pltpu
from jax.experimental.pallas import tpu_sc as plsc

N = 10000
D = 128
E = 320000
ROWS = 1000

NC = 2
NS = 16
NW = NC * NS
EPT = E // NW
CHUNK = 80
NCHUNK = EPT // CHUNK
SPLIT = 624
TAIL = N - SPLIT * NS
ZR = 16

_mesh = plsc.VectorSubcoreMesh(core_axis_name="c", subcore_axis_name="s",
                               num_cores=NC, num_subcores=NS)

def _agg_body(y_hbm, src_hbm, dst_hbm, out_hbm, src_v, dst_v, rows_v, zbuf_v, acc_sh, sem):
    c = lax.axis_index("c")
    s = lax.axis_index("s")
    wid = c * NS + s
    r0 = pl.multiple_of(s * SPLIT, 8)

    for r in range(ZR):
        for j in range(D // 16):
            zbuf_v[r, pl.ds(j * 16, 16)] = jnp.zeros((16,), jnp.float32)
    for k in range(SPLIT // ZR):
        pltpu.sync_copy(zbuf_v, acc_sh.at[pl.ds(r0 + k * ZR, ZR)])

    @pl.when(s == NS - 1)
    def _zero_tail():
        for k in range(TAIL // ZR):
            pltpu.sync_copy(zbuf_v, acc_sh.at[pl.ds(SPLIT * NS + k * ZR, ZR)])

    plsc.subcore_barrier()

    def chunk(i, _):
        off = wid * EPT + i * CHUNK
        pltpu.sync_copy(src_hbm.at[pl.ds(off, CHUNK)], src_v)
        pltpu.sync_copy(dst_hbm.at[pl.ds(off, CHUNK)], dst_v)
        pltpu.async_copy(y_hbm.at[src_v], rows_v, sem).wait()
        pltpu.sync_copy(rows_v, acc_sh.at[dst_v], add=True)
        return _

    lax.fori_loop(0, NCHUNK, chunk, None)
    plsc.subcore_barrier()

    pltpu.sync_copy(acc_sh.at[pl.ds(r0, SPLIT)], out_hbm.at[c, pl.ds(r0, SPLIT)])

    @pl.when(s == NS - 1)
    def _read_tail():
        pltpu.sync_copy(acc_sh.at[pl.ds(SPLIT * NS, TAIL)],
                        out_hbm.at[c, pl.ds(SPLIT * NS, TAIL)])


_sc_aggregate = functools.partial(
    pl.kernel,
    _agg_body,
    out_type=jax.ShapeDtypeStruct((NC, N, D), jnp.float32),
    mesh=_mesh,
    scratch_types=[
        pltpu.VMEM((CHUNK,), jnp.int32),
        pltpu.VMEM((CHUNK,), jnp.int32),
        pltpu.VMEM((CHUNK, D), jnp.float32),
        pltpu.VMEM((ZR, D), jnp.float32),
        pltpu.VMEM_SHARED((N, D), jnp.float32),
        pltpu.SemaphoreType.DMA,
    ],
)()


DW = 16


def _deg_body(dst_hbm, out_hbm, dst_v, ones_v, zbuf_v, acc_sh):
    c = lax.axis_index("c")
    s = lax.axis_index("s")
    wid = c * NS + s
    r0 = pl.multiple_of(s * SPLIT, 8)

    for r in range(CHUNK):
        ones_v[r, pl.ds(0, 16)] = jnp.ones((16,), jnp.float32)
    for r in range(ZR):
        zbuf_v[r, pl.ds(0, 16)] = jnp.zeros((16,), jnp.float32)
    for k in range(SPLIT // ZR):
        pltpu.sync_copy(zbuf_v, acc_sh.at[pl.ds(r0 + k * ZR, ZR)])

    @pl.when(s == NS - 1)
    def _zero_tail():
        for k in range(TAIL // ZR):
            pltpu.sync_copy(zbuf_v, acc_sh.at[pl.ds(SPLIT * NS + k * ZR, ZR)])

    plsc.subcore_barrier()

    def chunk(i, _):
        off = wid * EPT + i * CHUNK
        pltpu.sync_copy(dst_hbm.at[pl.ds(off, CHUNK)], dst_v)
        pltpu.sync_copy(ones_v, acc_sh.at[dst_v], add=True)
        return _

    lax.fori_loop(0, NCHUNK, chunk, None)
    plsc.subcore_barrier()

    pltpu.sync_copy(acc_sh.at[pl.ds(r0, SPLIT)], out_hbm.at[c, pl.ds(r0, SPLIT)])

    @pl.when(s == NS - 1)
    def _read_tail():
        pltpu.sync_copy(acc_sh.at[pl.ds(SPLIT * NS, TAIL)],
                        out_hbm.at[c, pl.ds(SPLIT * NS, TAIL)])


_sc_degree = functools.partial(
    pl.kernel,
    _deg_body,
    out_type=jax.ShapeDtypeStruct((NC, N, DW), jnp.float32),
    mesh=_mesh,
    scratch_types=[
        pltpu.VMEM((CHUNK,), jnp.int32),
        pltpu.VMEM((CHUNK, DW), jnp.float32),
        pltpu.VMEM((ZR, DW), jnp.float32),
        pltpu.VMEM_SHARED((N, DW), jnp.float32),
    ],
)()


def _first_layer_body(deg_ref, x_ref, w_ref, y_ref, dinv_ref):
    deg = deg_ref[0, :, 0:1] + deg_ref[1, :, 0:1] + 1.0
    dinv = lax.rsqrt(deg)
    dinv_ref[...] = dinv
    y_ref[...] = jnp.dot(x_ref[...] * dinv, w_ref[...],
                         preferred_element_type=jnp.float32)


def _first_layer(degp, x, W1):
    grid = (N // ROWS,)
    return pl.pallas_call(
        _first_layer_body,
        grid=grid,
        in_specs=[
            pl.BlockSpec((2, ROWS, DW), lambda i: (0, i, 0)),
            pl.BlockSpec((ROWS, D), lambda i: (i, 0)),
            pl.BlockSpec((D, D), lambda i: (0, 0)),
        ],
        out_specs=[
            pl.BlockSpec((ROWS, D), lambda i: (i, 0)),
            pl.BlockSpec((ROWS, 1), lambda i: (i, 0)),
        ],
        out_shape=[
            jax.ShapeDtypeStruct((N, D), jnp.float32),
            jax.ShapeDtypeStruct((N, 1), jnp.float32),
        ],
    )(degp, x, W1)


def _mid_layer_body(agg_ref, y_ref, dinv_ref, b_ref, w_ref, ynext_ref):
    dinv = dinv_ref[...]
    agg = agg_ref[0] + agg_ref[1] + y_ref[...]
    out = dinv * agg + b_ref[...]
    xn = jnp.where(out > 0, out, jnp.exp(jnp.minimum(out, 0.0)) - 1.0)
    ynext_ref[...] = jnp.dot(xn * dinv, w_ref[...],
                             preferred_element_type=jnp.float32)


def _mid_layer(agg, y, dinv, b, Wn):
    grid = (N // ROWS,)
    return pl.pallas_call(
        _mid_layer_body,
        grid=grid,
        in_specs=[
            pl.BlockSpec((2, ROWS, D), lambda i: (0, i, 0)),
            pl.BlockSpec((ROWS, D), lambda i: (i, 0)),
            pl.BlockSpec((ROWS, 1), lambda i: (i, 0)),
            pl.BlockSpec((1, D), lambda i: (0, 0)),
            pl.BlockSpec((D, D), lambda i: (0, 0)),
        ],
        out_specs=pl.BlockSpec((ROWS, D), lambda i: (i, 0)),
        out_shape=jax.ShapeDtypeStruct((N, D), jnp.float32),
    )(agg, y, dinv, b, Wn)


def _last_layer_body(agg_ref, y_ref, dinv_ref, b_ref, out_ref):
    agg = agg_ref[0] + agg_ref[1] + y_ref[...]
    out_ref[...] = dinv_ref[...] * agg + b_ref[...]


def _last_layer(agg, y, dinv, b):
    grid = (N // ROWS,)
    return pl.pallas_call(
        _last_layer_body,
        grid=grid,
        in_specs=[
            pl.BlockSpec((2, ROWS, D), lambda i: (0, i, 0)),
            pl.BlockSpec((ROWS, D), lambda i: (i, 0)),
            pl.BlockSpec((ROWS, 1), lambda i: (i, 0)),
            pl.BlockSpec((1, D), lambda i: (0, 0)),
        ],
        out_specs=pl.BlockSpec((ROWS, D), lambda i: (i, 0)),
        out_shape=jax.ShapeDtypeStruct((N, D), jnp.float32),
    )(agg, y, dinv, b)


@jax.jit
def kernel(x, edge_index, W1, b1, W2, b2, W3, b3):
    src_idx = edge_index[0]
    dst_idx = edge_index[1]
    degp = _sc_degree(dst_idx)
    y1, dinv = _first_layer(degp, x, W1)
    a1 = _sc_aggregate(y1, src_idx, dst_idx)
    y2 = _mid_layer(a1, y1, dinv, b1.reshape(1, D), W2)
    a2 = _sc_aggregate(y2, src_idx, dst_idx)
    y3 = _mid_layer(a2, y2, dinv, b2.reshape(1, D), W3)
    a3 = _sc_aggregate(y3, src_idx, dst_idx)
    return _last_layer(a3, y3, dinv, b3.reshape(1, D))
